# baseline (speedup 1.0000x reference)
import functools

import jax
import jax.numpy as jnp
from jax import lax
from jax.experimental import pallas as pl
from jax.experimental.pallas import tpu as pltpu
from jax.experimental.pallas import tpu_sc as plsc

B, S, D = 4, 2048, 768
N = B * S
NH = N // 2
NW = 32
TPW = NH // NW
CH = 64
TB = 1024
SB = S // TB
BH = B // 2


def _sc_gather_body(ids_hbm, tok_hbm, out_hbm, idx0, idx1, rows0, rows1,
                    sem0, sem1):
    c = lax.axis_index("c")
    s = lax.axis_index("s")
    w = s * 2 + c
    nk = TPW // CH
    idx = (idx0, idx1)
    rows = (rows0, rows1)
    sems = (sem0, sem1)
    copies = [None] * nk

    def start(k):
        base = w * TPW + k * CH
        pltpu.sync_copy(ids_hbm.at[pl.ds(base, CH)], idx[k % 2])
        copies[k] = pltpu.async_copy(tok_hbm.at[idx[k % 2]], rows[k % 2],
                                     sems[k % 2])

    start(0)
    if nk > 1:
        start(1)
    for k in range(nk):
        base = w * TPW + k * CH
        copies[k].wait()
        pltpu.sync_copy(rows[k % 2], out_hbm.at[pl.ds(base, CH), :])
        if k + 2 < nk:
            start(k + 2)


def _sc_gather(ids_half, token_table):
    mesh = plsc.VectorSubcoreMesh(core_axis_name="c", subcore_axis_name="s")
    call = functools.partial(
        pl.kernel,
        mesh=mesh,
        out_type=jax.ShapeDtypeStruct((NH, D), jnp.float32),
        scratch_types=[
            pltpu.VMEM((CH,), jnp.int32),
            pltpu.VMEM((CH,), jnp.int32),
            pltpu.VMEM((CH, D), jnp.float32),
            pltpu.VMEM((CH, D), jnp.float32),
            pltpu.SemaphoreType.DMA,
            pltpu.SemaphoreType.DMA,
        ],
    )(_sc_gather_body)
    return call(ids_half, token_table)


def _tc_ln_body(rows_ref, ttf_ref, pos_ref, type_ref, gamma_ref, beta_ref,
                *rest):
    out_ref = rest[-1]
    x = rows_ref[...] + pos_ref[...]
    ttf = ttf_ref[...]
    t0 = type_ref[0:1, :]
    t1 = type_ref[1:2, :]
    x = x + t0 + ttf * (t1 - t0)
    mean = jnp.mean(x, axis=1, keepdims=True)
    xc = x - mean
    var = jnp.mean(xc * xc, axis=1, keepdims=True)
    inv = lax.rsqrt(var + jnp.float32(1e-12))
    out_ref[...] = xc * inv * gamma_ref[...] + beta_ref[...]


def _tc_ln_half(rows_half, ttf_half, pos_table, type_table, gamma2, beta2,
                prev_out, half):
    in_specs = [
        pl.BlockSpec((TB, D), lambda k, b: (b * SB + k, 0)),
        pl.BlockSpec((TB, 1), lambda k, b: (b * SB + k, 0)),
        pl.BlockSpec((TB, D), lambda k, b: (k, 0)),
        pl.BlockSpec((2, D), lambda k, b: (0, 0)),
        pl.BlockSpec((1, D), lambda k, b: (0, 0)),
        pl.BlockSpec((1, D), lambda k, b: (0, 0)),
    ]
    args = [rows_half, ttf_half, pos_table, type_table, gamma2, beta2]
    aliases = {}
    if prev_out is not None:
        in_specs.append(pl.BlockSpec(
            (TB, D), lambda k, b, h=half: ((h * BH + b) * SB + k, 0)))
        args.append(prev_out)
        aliases = {6: 0}
    return pl.pallas_call(
        _tc_ln_body,
        grid=(SB, BH),
        in_specs=in_specs,
        out_specs=pl.BlockSpec(
            (TB, D), lambda k, b, h=half: ((h * BH + b) * SB + k, 0)),
        out_shape=jax.ShapeDtypeStruct((N, D), jnp.float32),
        input_output_aliases=aliases,
    )(*args)


def kernel(input_ids, token_type_ids, token_table, pos_table, type_table,
           gamma, beta):
    ids = input_ids.reshape(-1).astype(jnp.int32)
    ttf = token_type_ids.reshape(-1, 1).astype(jnp.float32)
    gamma2 = gamma.reshape(1, D)
    beta2 = beta.reshape(1, D)

    rows0 = _sc_gather(ids[:NH], token_table)
    rows1 = _sc_gather(ids[NH:], token_table)

    out = _tc_ln_half(rows0, ttf[:NH], pos_table, type_table, gamma2, beta2,
                      None, 0)
    out = _tc_ln_half(rows1, ttf[NH:], pos_table, type_table, gamma2, beta2,
                      out, 1)
    return out.reshape(B, S, D)


# speedup vs baseline: 1.0367x; 1.0367x over previous
import functools

import jax
import jax.numpy as jnp
from jax import lax
from jax.experimental import pallas as pl
from jax.experimental.pallas import tpu as pltpu
from jax.experimental.pallas import tpu_sc as plsc

B, S, D = 4, 2048, 768
N = B * S
SH = S // 2
NH = B * SH
NW = 32
WPB = NW // B
TPW = NH // NW
CH = 64
TB = 1024


def _sc_gather_body(half, ids_hbm, tok_hbm, out_hbm, idx0, idx1, rows0,
                    rows1, sem0, sem1):
    c = lax.axis_index("c")
    s = lax.axis_index("s")
    w = s * 2 + c
    b = w // WPB
    soff = half * SH + (w % WPB) * TPW
    src_base = b * S + soff
    dst_base = w * TPW
    nk = TPW // CH
    idx = (idx0, idx1)
    rows = (rows0, rows1)
    sems = (sem0, sem1)
    copies = [None] * nk

    def start(k):
        pltpu.sync_copy(ids_hbm.at[pl.ds(src_base + k * CH, CH)], idx[k % 2])
        copies[k] = pltpu.async_copy(tok_hbm.at[idx[k % 2]], rows[k % 2],
                                     sems[k % 2])

    start(0)
    if nk > 1:
        start(1)
    for k in range(nk):
        copies[k].wait()
        pltpu.sync_copy(rows[k % 2],
                        out_hbm.at[pl.ds(dst_base + k * CH, CH), :])
        if k + 2 < nk:
            start(k + 2)


def _sc_gather(ids, token_table, half):
    mesh = plsc.VectorSubcoreMesh(core_axis_name="c", subcore_axis_name="s")
    call = functools.partial(
        pl.kernel,
        mesh=mesh,
        out_type=jax.ShapeDtypeStruct((NH, D), jnp.float32),
        scratch_types=[
            pltpu.VMEM((CH,), jnp.int32),
            pltpu.VMEM((CH,), jnp.int32),
            pltpu.VMEM((CH, D), jnp.float32),
            pltpu.VMEM((CH, D), jnp.float32),
            pltpu.SemaphoreType.DMA,
            pltpu.SemaphoreType.DMA,
        ],
    )(functools.partial(_sc_gather_body, half))
    return call(ids, token_table)


def _tc_ln_body(rows_ref, ttf_ref, pos_ref, type_ref, gamma_ref, beta_ref,
                *rest):
    out_ref = rest[-1]
    x = rows_ref[...] + pos_ref[...]
    ttf = ttf_ref[...]
    t0 = type_ref[0:1, :]
    t1 = type_ref[1:2, :]
    x = x + t0 + ttf * (t1 - t0)
    mean = jnp.mean(x, axis=1, keepdims=True)
    xc = x - mean
    var = jnp.mean(xc * xc, axis=1, keepdims=True)
    inv = lax.rsqrt(var + jnp.float32(1e-12))
    out_ref[...] = xc * inv * gamma_ref[...] + beta_ref[...]


def _tc_ln_half(rows_half, ttf, pos_table, type_table, gamma2, beta2,
                prev_out, half):
    in_specs = [
        pl.BlockSpec((TB, D), lambda b: (b, 0)),
        pl.BlockSpec((TB, 1), lambda b, h=half: (2 * b + h, 0)),
        pl.BlockSpec((TB, D), lambda b, h=half: (h, 0)),
        pl.BlockSpec((2, D), lambda b: (0, 0)),
        pl.BlockSpec((1, D), lambda b: (0, 0)),
        pl.BlockSpec((1, D), lambda b: (0, 0)),
    ]
    args = [rows_half, ttf, pos_table, type_table, gamma2, beta2]
    aliases = {}
    if prev_out is not None:
        in_specs.append(
            pl.BlockSpec((TB, D), lambda b, h=half: (2 * b + h, 0)))
        args.append(prev_out)
        aliases = {6: 0}
    return pl.pallas_call(
        _tc_ln_body,
        grid=(B,),
        in_specs=in_specs,
        out_specs=pl.BlockSpec((TB, D), lambda b, h=half: (2 * b + h, 0)),
        out_shape=jax.ShapeDtypeStruct((N, D), jnp.float32),
        input_output_aliases=aliases,
    )(*args)


def kernel(input_ids, token_type_ids, token_table, pos_table, type_table,
           gamma, beta):
    ids = input_ids.reshape(-1).astype(jnp.int32)
    ttf = token_type_ids.reshape(-1, 1).astype(jnp.float32)
    gamma2 = gamma.reshape(1, D)
    beta2 = beta.reshape(1, D)

    rows0 = _sc_gather(ids, token_table, 0)
    rows1 = _sc_gather(ids, token_table, 1)

    out = _tc_ln_half(rows0, ttf, pos_table, type_table, gamma2, beta2,
                      None, 0)
    out = _tc_ln_half(rows1, ttf, pos_table, type_table, gamma2, beta2,
                      out, 1)
    return out.reshape(B, S, D)


# speedup vs baseline: 1.0440x; 1.0071x over previous
import functools

import jax
import jax.numpy as jnp
from jax import lax
from jax.experimental import pallas as pl
from jax.experimental.pallas import tpu as pltpu
from jax.experimental.pallas import tpu_sc as plsc

B, S, D = 4, 2048, 768
N = B * S
BH = B // 2
NH = BH * S
NW = 32
TPW = NH // NW
CH = 64


def _sc_gather_body(half, ids_hbm, tok_hbm, out_hbm, idx0, idx1, rows0,
                    rows1, sem0, sem1):
    c = lax.axis_index("c")
    s = lax.axis_index("s")
    w = s * 2 + c
    src_base = half * NH + w * TPW
    dst_base = w * TPW
    nk = TPW // CH
    idx = (idx0, idx1)
    rows = (rows0, rows1)
    sems = (sem0, sem1)
    copies = [None] * nk

    def start(k):
        pltpu.sync_copy(ids_hbm.at[pl.ds(src_base + k * CH, CH)], idx[k % 2])
        copies[k] = pltpu.async_copy(tok_hbm.at[idx[k % 2]], rows[k % 2],
                                     sems[k % 2])

    start(0)
    if nk > 1:
        start(1)
    for k in range(nk):
        copies[k].wait()
        pltpu.sync_copy(rows[k % 2],
                        out_hbm.at[pl.ds(dst_base + k * CH, CH), :])
        if k + 2 < nk:
            start(k + 2)


def _sc_gather(ids, token_table, half):
    mesh = plsc.VectorSubcoreMesh(core_axis_name="c", subcore_axis_name="s")
    call = functools.partial(
        pl.kernel,
        mesh=mesh,
        out_type=jax.ShapeDtypeStruct((NH, D), jnp.float32),
        scratch_types=[
            pltpu.VMEM((CH,), jnp.int32),
            pltpu.VMEM((CH,), jnp.int32),
            pltpu.VMEM((CH, D), jnp.float32),
            pltpu.VMEM((CH, D), jnp.float32),
            pltpu.SemaphoreType.DMA,
            pltpu.SemaphoreType.DMA,
        ],
    )(functools.partial(_sc_gather_body, half))
    return call(ids, token_table)


def _tc_ln_body(rows_ref, ttf_ref, pos_ref, type_ref, gamma_ref, beta_ref,
                *rest):
    out_ref = rest[-1]
    x = rows_ref[...] + pos_ref[...]
    ttf = ttf_ref[...]
    t0 = type_ref[0:1, :]
    t1 = type_ref[1:2, :]
    x = x + t0 + ttf * (t1 - t0)
    mean = jnp.mean(x, axis=1, keepdims=True)
    xc = x - mean
    var = jnp.mean(xc * xc, axis=1, keepdims=True)
    inv = lax.rsqrt(var + jnp.float32(1e-12))
    out_ref[...] = xc * inv * gamma_ref[...] + beta_ref[...]


def _tc_ln_half(rows_half, ttf, pos_table, type_table, gamma2, beta2,
                prev_out, half):
    in_specs = [
        pl.BlockSpec((S, D), lambda b: (b, 0)),
        pl.BlockSpec((S, 1), lambda b, h=half: (BH * h + b, 0)),
        pl.BlockSpec((S, D), lambda b: (0, 0)),
        pl.BlockSpec((2, D), lambda b: (0, 0)),
        pl.BlockSpec((1, D), lambda b: (0, 0)),
        pl.BlockSpec((1, D), lambda b: (0, 0)),
    ]
    args = [rows_half, ttf, pos_table, type_table, gamma2, beta2]
    aliases = {}
    if prev_out is not None:
        in_specs.append(
            pl.BlockSpec((S, D), lambda b, h=half: (BH * h + b, 0)))
        args.append(prev_out)
        aliases = {6: 0}
    return pl.pallas_call(
        _tc_ln_body,
        grid=(BH,),
        in_specs=in_specs,
        out_specs=pl.BlockSpec((S, D), lambda b, h=half: (BH * h + b, 0)),
        out_shape=jax.ShapeDtypeStruct((N, D), jnp.float32),
        input_output_aliases=aliases,
    )(*args)


def kernel(input_ids, token_type_ids, token_table, pos_table, type_table,
           gamma, beta):
    ids = input_ids.reshape(-1).astype(jnp.int32)
    ttf = token_type_ids.reshape(-1, 1).astype(jnp.float32)
    gamma2 = gamma.reshape(1, D)
    beta2 = beta.reshape(1, D)

    rows0 = _sc_gather(ids, token_table, 0)
    rows1 = _sc_gather(ids, token_table, 1)

    out = _tc_ln_half(rows0, ttf, pos_table, type_table, gamma2, beta2,
                      None, 0)
    out = _tc_ln_half(rows1, ttf, pos_table, type_table, gamma2, beta2,
                      out, 1)
    return out.reshape(B, S, D)


# speedup vs baseline: 1.0677x; 1.0227x over previous
import functools

import jax
import jax.numpy as jnp
from jax import lax
from jax.experimental import pallas as pl
from jax.experimental.pallas import tpu as pltpu
from jax.experimental.pallas import tpu_sc as plsc

B, S, D = 4, 2048, 768
N = B * S
BH = B // 2
NH = BH * S
NW = 32
TPW = NH // NW
CH = 64


def _sc_gather_body(half, ids_hbm, tok_hbm, out_hbm, idx0, idx1, rows0,
                    rows1, sem0, sem1):
    c = lax.axis_index("c")
    s = lax.axis_index("s")
    w = s * 2 + c
    b = half * BH + w // (NW // BH)
    soff = (w % (NW // BH)) * TPW
    dst_base = w * TPW
    nk = TPW // CH
    idx = (idx0, idx1)
    rows = (rows0, rows1)
    sems = (sem0, sem1)
    copies = [None] * nk

    def start(k):
        pltpu.sync_copy(ids_hbm.at[b, pl.ds(soff + k * CH, CH)], idx[k % 2])
        copies[k] = pltpu.async_copy(tok_hbm.at[idx[k % 2]], rows[k % 2],
                                     sems[k % 2])

    start(0)
    if nk > 1:
        start(1)
    for k in range(nk):
        copies[k].wait()
        pltpu.sync_copy(rows[k % 2],
                        out_hbm.at[pl.ds(dst_base + k * CH, CH), :])
        if k + 2 < nk:
            start(k + 2)


def _sc_gather(ids, token_table, half):
    mesh = plsc.VectorSubcoreMesh(core_axis_name="c", subcore_axis_name="s")
    call = functools.partial(
        pl.kernel,
        mesh=mesh,
        out_type=jax.ShapeDtypeStruct((NH, D), jnp.float32),
        scratch_types=[
            pltpu.VMEM((CH,), jnp.int32),
            pltpu.VMEM((CH,), jnp.int32),
            pltpu.VMEM((CH, D), jnp.float32),
            pltpu.VMEM((CH, D), jnp.float32),
            pltpu.SemaphoreType.DMA,
            pltpu.SemaphoreType.DMA,
        ],
    )(functools.partial(_sc_gather_body, half))
    return call(ids, token_table)


def _tc_ln_body(rows_ref, tt_ref, pos_ref, type_ref, gamma_ref, beta_ref,
                *rest):
    out_ref = rest[-1]
    x = rows_ref[...] + pos_ref[...]
    ttf = tt_ref[...].reshape(S, 1).astype(jnp.float32)
    t0 = type_ref[0:1, :]
    t1 = type_ref[1:2, :]
    x = x + t0 + ttf * (t1 - t0)
    mean = jnp.mean(x, axis=1, keepdims=True)
    xc = x - mean
    var = jnp.mean(xc * xc, axis=1, keepdims=True)
    inv = lax.rsqrt(var + jnp.float32(1e-12))
    out_ref[...] = xc * inv * gamma_ref[...] + beta_ref[...]


def _tc_ln_half(rows_half, ttf, pos_table, type_table, gamma2, beta2,
                prev_out, half):
    in_specs = [
        pl.BlockSpec((S, D), lambda b: (b, 0)),
        pl.BlockSpec((1, 1, S), lambda b, h=half: (BH * h + b, 0, 0)),
        pl.BlockSpec((S, D), lambda b: (0, 0)),
        pl.BlockSpec((2, D), lambda b: (0, 0)),
        pl.BlockSpec((1, D), lambda b: (0, 0)),
        pl.BlockSpec((1, D), lambda b: (0, 0)),
    ]
    args = [rows_half, ttf, pos_table, type_table, gamma2, beta2]
    aliases = {}
    if prev_out is not None:
        in_specs.append(
            pl.BlockSpec((S, D), lambda b, h=half: (BH * h + b, 0)))
        args.append(prev_out)
        aliases = {6: 0}
    return pl.pallas_call(
        _tc_ln_body,
        grid=(BH,),
        in_specs=in_specs,
        out_specs=pl.BlockSpec((S, D), lambda b, h=half: (BH * h + b, 0)),
        out_shape=jax.ShapeDtypeStruct((N, D), jnp.float32),
        input_output_aliases=aliases,
    )(*args)


def kernel(input_ids, token_type_ids, token_table, pos_table, type_table,
           gamma, beta):
    ids = input_ids.astype(jnp.int32)
    tt = token_type_ids.astype(jnp.int32).reshape(B, 1, S)
    gamma2 = gamma.reshape(1, D)
    beta2 = beta.reshape(1, D)

    rows0 = _sc_gather(ids, token_table, 0)
    rows1 = _sc_gather(ids, token_table, 1)

    out = _tc_ln_half(rows0, tt, pos_table, type_table, gamma2, beta2,
                      None, 0)
    out = _tc_ln_half(rows1, tt, pos_table, type_table, gamma2, beta2,
                      out, 1)
    return out.reshape(B, S, D)


# speedup vs baseline: 1.0703x; 1.0024x over previous
import functools

import jax
import jax.numpy as jnp
from jax import lax
from jax.experimental import pallas as pl
from jax.experimental.pallas import tpu as pltpu
from jax.experimental.pallas import tpu_sc as plsc

B, S, D = 4, 2048, 768
N = B * S
BH = B // 2
NH = BH * S
NW = 32
TPW = NH // NW
CH = 64


def _sc_gather_body(ids_hbm, tok_hbm, out_hbm, idx0, idx1, rows0,
                    rows1, sem0, sem1):
    c = lax.axis_index("c")
    s = lax.axis_index("s")
    w = s * 2 + c
    b = w // (NW // BH)
    soff = (w % (NW // BH)) * TPW
    dst_base = w * TPW
    nk = TPW // CH
    idx = (idx0, idx1)
    rows = (rows0, rows1)
    sems = (sem0, sem1)
    copies = [None] * nk

    def start(k):
        pltpu.sync_copy(ids_hbm.at[b, pl.ds(soff + k * CH, CH)], idx[k % 2])
        copies[k] = pltpu.async_copy(tok_hbm.at[idx[k % 2]], rows[k % 2],
                                     sems[k % 2])

    start(0)
    if nk > 1:
        start(1)
    for k in range(nk):
        copies[k].wait()
        pltpu.sync_copy(rows[k % 2],
                        out_hbm.at[pl.ds(dst_base + k * CH, CH), :])
        if k + 2 < nk:
            start(k + 2)


def _sc_gather(ids_half, token_table):
    mesh = plsc.VectorSubcoreMesh(core_axis_name="c", subcore_axis_name="s")
    call = functools.partial(
        pl.kernel,
        mesh=mesh,
        out_type=jax.ShapeDtypeStruct((NH, D), jnp.float32),
        scratch_types=[
            pltpu.VMEM((CH,), jnp.int32),
            pltpu.VMEM((CH,), jnp.int32),
            pltpu.VMEM((CH, D), jnp.float32),
            pltpu.VMEM((CH, D), jnp.float32),
            pltpu.SemaphoreType.DMA,
            pltpu.SemaphoreType.DMA,
        ],
    )(_sc_gather_body)
    return call(ids_half, token_table)


def _tc_ln_body(rows_ref, tt_ref, pos_ref, type_ref, gamma_ref, beta_ref,
                *rest):
    out_ref = rest[-1]
    x = rows_ref[...] + pos_ref[...]
    ttf = tt_ref[...].reshape(S, 1).astype(jnp.float32)
    t0 = type_ref[0:1, :]
    t1 = type_ref[1:2, :]
    x = x + t0 + ttf * (t1 - t0)
    mean = jnp.mean(x, axis=1, keepdims=True)
    xc = x - mean
    var = jnp.mean(xc * xc, axis=1, keepdims=True)
    inv = lax.rsqrt(var + jnp.float32(1e-12))
    out_ref[...] = xc * inv * gamma_ref[...] + beta_ref[...]


def _tc_ln_half(rows_half, ttf, pos_table, type_table, gamma2, beta2,
                prev_out, half):
    in_specs = [
        pl.BlockSpec((S, D), lambda b: (b, 0)),
        pl.BlockSpec((1, 1, S), lambda b, h=half: (BH * h + b, 0, 0)),
        pl.BlockSpec((S, D), lambda b: (0, 0)),
        pl.BlockSpec((2, D), lambda b: (0, 0)),
        pl.BlockSpec((1, D), lambda b: (0, 0)),
        pl.BlockSpec((1, D), lambda b: (0, 0)),
    ]
    args = [rows_half, ttf, pos_table, type_table, gamma2, beta2]
    aliases = {}
    if prev_out is not None:
        in_specs.append(
            pl.BlockSpec((S, D), lambda b, h=half: (BH * h + b, 0)))
        args.append(prev_out)
        aliases = {6: 0}
    return pl.pallas_call(
        _tc_ln_body,
        grid=(BH,),
        in_specs=in_specs,
        out_specs=pl.BlockSpec((S, D), lambda b, h=half: (BH * h + b, 0)),
        out_shape=jax.ShapeDtypeStruct((N, D), jnp.float32),
        input_output_aliases=aliases,
    )(*args)


def kernel(input_ids, token_type_ids, token_table, pos_table, type_table,
           gamma, beta):
    ids = input_ids.astype(jnp.int32)
    tt = token_type_ids.astype(jnp.int32).reshape(B, 1, S)
    gamma2 = gamma.reshape(1, D)
    beta2 = beta.reshape(1, D)

    rows0 = _sc_gather(ids[:BH], token_table)
    rows1 = _sc_gather(ids[BH:], token_table)

    out = _tc_ln_half(rows0, tt, pos_table, type_table, gamma2, beta2,
                      None, 0)
    out = _tc_ln_half(rows1, tt, pos_table, type_table, gamma2, beta2,
                      out, 1)
    return out.reshape(B, S, D)
